# baseline (speedup 1.0000x reference)
import functools

import jax
import jax.numpy as jnp
from jax import lax
from jax.experimental import pallas as pl
from jax.experimental.pallas import tpu as pltpu
from jax.experimental.pallas import tpu_sc as plsc

_N = 10000
_HID = 128
_E = 320000
_NC = 2
_NS = 16
_NW = _NC * _NS
_LANES = 128
_KG = 10
_GI = 8
_K = _KG * _GI
_E_PAD = _NW * _K * _LANES
_FAST_CID = 1
_KF = 144
_KS = 16
_KGF = _KF // _GI
_KGS = _KS // _GI
_SLOW_CID = 1 - _FAST_CID
_K2 = 2 * _K
_KG2 = _K2 // _GI
_N_PAD = 10240
_RPT = _N_PAD // _NS


def _tc_prep(x_ref, wh_ref, bh_ref, t0_ref):
    wbar = jnp.mean(wh_ref[...], axis=0)
    bbar = jnp.mean(bh_ref[...], axis=0, keepdims=True)
    t0_ref[...] = lax.dot_general(
        x_ref[...], wbar, (((1,), (1,)), ((), ())),
        preferred_element_type=jnp.float32,
        precision=lax.Precision.HIGHEST) + bbar


def _tc_mid(part_ref, wh_ref, bh_ref, h1_ref, t1_ref):
    deg = part_ref[_SLOW_CID, :, 0:1]
    den = jnp.maximum(deg, 1.0)
    h1 = part_ref[_FAST_CID] / den
    wbar = jnp.mean(wh_ref[...], axis=0)
    bbar = jnp.mean(bh_ref[...], axis=0, keepdims=True)
    h1_ref[...] = h1
    t1_ref[...] = lax.dot_general(
        h1, wbar, (((1,), (1,)), ((), ())),
        preferred_element_type=jnp.float32,
        precision=lax.Precision.HIGHEST) + bbar


def _tc_final(part2_ref, degp_ref, h1_ref, hw_ref, wp_ref, bp_ref, gv_ref,
              out_ref):
    deg = degp_ref[_SLOW_CID, :, 0:1]
    den = jnp.maximum(deg, 1.0)
    h2 = (part2_ref[0] + part2_ref[1]) / den
    ehw = jnp.exp(hw_ref[...])
    hws = jnp.sum(ehw)
    hf = h1_ref[...] * (ehw[:, 0:1] / hws) + h2 * (ehw[:, 1:2] / hws)
    egv = jnp.exp(gv_ref[...])
    gw = egv / jnp.sum(egv)
    col = lax.broadcasted_iota(jnp.int32, (8, _HID), 1) // 16
    row = lax.broadcasted_iota(jnp.int32, (8, _HID), 0)
    sel = (col == row).astype(jnp.float32)
    gwv = jnp.dot(gw, sel, preferred_element_type=jnp.float32)
    z = lax.dot_general(
        hf, wp_ref[...], (((1,), (1,)), ((), ())),
        preferred_element_type=jnp.float32,
        precision=lax.Precision.HIGHEST) + bp_ref[...]
    out_ref[...] = z * gwv


def _sc_edge_body(table_hbm, srcr_hbm, dstr_hbm, zrows_hbm, part_hbm,
                  acc, src_v, dst_v, rows_a, rows_b, sem_a, sem_b):
    cid = lax.axis_index("c")
    sid = lax.axis_index("s")
    row0 = sid * _RPT

    pltpu.sync_copy(zrows_hbm, rows_a)
    for j in range(_RPT // _LANES):
        pltpu.sync_copy(rows_a, acc.at[pl.ds(row0 + j * _LANES, _LANES)])
    plsc.subcore_barrier()

    bufs = (rows_a, rows_b)
    sems = (sem_a, sem_b)

    def group(g, carry):
        pltpu.sync_copy(srcr_hbm.at[cid, sid, pl.ds(g * _GI, _GI)], src_v)
        pltpu.sync_copy(dstr_hbm.at[cid, sid, pl.ds(g * _GI, _GI)], dst_v)
        cps = {0: pltpu.async_copy(table_hbm.at[src_v.at[0]], bufs[0],
                                   sems[0])}
        for j in range(_GI):
            cps[j].wait()
            if j + 1 < _GI:
                cps[j + 1] = pltpu.async_copy(
                    table_hbm.at[src_v.at[j + 1]], bufs[(j + 1) % 2],
                    sems[(j + 1) % 2])
            pltpu.sync_copy(bufs[j % 2], acc.at[dst_v.at[j]], add=True)
        return carry

    kg = jnp.where(cid == _FAST_CID, _KGF, _KGS)
    lax.fori_loop(0, kg, group, 0)

    plsc.subcore_barrier()
    for j in range(_RPT // _LANES):
        r = row0 + j * _LANES
        pltpu.sync_copy(acc.at[pl.ds(r, _LANES)], rows_a)
        pltpu.sync_copy(rows_a, part_hbm.at[cid, pl.ds(r, _LANES)])


@functools.lru_cache(maxsize=None)
def _build_sc_pass():
    return pl.kernel(
        _sc_edge_body,
        mesh=plsc.VectorSubcoreMesh(core_axis_name="c", subcore_axis_name="s"),
        out_type=jax.ShapeDtypeStruct((_NC, _N_PAD, _HID), jnp.float32),
        scratch_types=[
            pltpu.VMEM_SHARED((_N_PAD, _HID), jnp.float32),
            pltpu.VMEM((_GI, _LANES), jnp.int32),
            pltpu.VMEM((_GI, _LANES), jnp.int32),
            pltpu.VMEM((_LANES, _HID), jnp.float32),
            pltpu.VMEM((_LANES, _HID), jnp.float32),
            pltpu.SemaphoreType.DMA,
            pltpu.SemaphoreType.DMA,
        ],
    )


def _sc_hop1_body(table_hbm, srcr2_hbm, dstr2_hbm, zrows_hbm, ones_hbm,
                  part_hbm, acc, src_v, dst_v, rows_a, rows_b, sem_a, sem_b):
    cid = lax.axis_index("c")
    sid = lax.axis_index("s")
    row0 = sid * _RPT

    pltpu.sync_copy(zrows_hbm, rows_a)
    for j in range(_RPT // _LANES):
        pltpu.sync_copy(rows_a, acc.at[pl.ds(row0 + j * _LANES, _LANES)])
    plsc.subcore_barrier()

    bufs = (rows_a, rows_b)
    sems = (sem_a, sem_b)

    @pl.when(cid == _FAST_CID)
    def _fast():
        def group(g, carry):
            pltpu.sync_copy(srcr2_hbm.at[sid, pl.ds(g * _GI, _GI)], src_v)
            pltpu.sync_copy(dstr2_hbm.at[sid, pl.ds(g * _GI, _GI)], dst_v)
            cps = {0: pltpu.async_copy(table_hbm.at[src_v.at[0]], bufs[0],
                                       sems[0])}
            for j in range(_GI):
                cps[j].wait()
                if j + 1 < _GI:
                    cps[j + 1] = pltpu.async_copy(
                        table_hbm.at[src_v.at[j + 1]], bufs[(j + 1) % 2],
                        sems[(j + 1) % 2])
                pltpu.sync_copy(bufs[j % 2], acc.at[dst_v.at[j]], add=True)
            return carry

        lax.fori_loop(0, _KG2, group, 0)

    @pl.when(cid != _FAST_CID)
    def _slow():
        pltpu.sync_copy(ones_hbm, rows_a)

        def group(g, carry):
            pltpu.sync_copy(dstr2_hbm.at[sid, pl.ds(g * _GI, _GI)], dst_v)
            for j in range(_GI):
                pltpu.sync_copy(rows_a, acc.at[dst_v.at[j]], add=True)
            return carry

        lax.fori_loop(0, _KG2, group, 0)

    plsc.subcore_barrier()
    for j in range(_RPT // _LANES):
        r = row0 + j * _LANES
        pltpu.sync_copy(acc.at[pl.ds(r, _LANES)], rows_b)
        pltpu.sync_copy(rows_b, part_hbm.at[cid, pl.ds(r, _LANES)])


@functools.lru_cache(maxsize=None)
def _build_sc_hop1():
    return pl.kernel(
        _sc_hop1_body,
        mesh=plsc.VectorSubcoreMesh(core_axis_name="c", subcore_axis_name="s"),
        out_type=jax.ShapeDtypeStruct((_NC, _N_PAD, _HID), jnp.float32),
        scratch_types=[
            pltpu.VMEM_SHARED((_N_PAD, _HID), jnp.float32),
            pltpu.VMEM((_GI, _LANES), jnp.int32),
            pltpu.VMEM((_GI, _LANES), jnp.int32),
            pltpu.VMEM((_LANES, _HID), jnp.float32),
            pltpu.VMEM((_LANES, _HID), jnp.float32),
            pltpu.SemaphoreType.DMA,
            pltpu.SemaphoreType.DMA,
        ],
    )


def kernel(node_features, edge_index, W_hop, b_hop, hop_weights, W_perc,
           b_perc, gate_vector):
    f32 = jnp.float32
    x = jnp.pad(node_features, ((0, _N_PAD - _N), (0, 0)))
    src = edge_index[0]
    dst = edge_index[1]
    pad_e = _E_PAD - _E
    src_p = jnp.concatenate([src, jnp.zeros((pad_e,), jnp.int32)])
    dst_p = jnp.concatenate([dst, jnp.full((pad_e,), _N, jnp.int32)])
    srcr2 = src_p.reshape(_NS, _K2, _LANES)
    dstr2 = dst_p.reshape(_NS, _K2, _LANES)

    def _split(e_p):
        ef = _NS * _KF * _LANES
        fast = e_p[:ef].reshape(_NS, _KF, _LANES)
        slow = jnp.pad(
            e_p[ef:].reshape(_NS, _KS, _LANES),
            ((0, 0), (0, _KF - _KS), (0, 0)))
        cores = [None, None]
        cores[_FAST_CID] = fast
        cores[1 - _FAST_CID] = slow
        return jnp.stack(cores)
    srcr = _split(src_p)
    dstr = _split(dst_p)
    zrows = jnp.zeros((_LANES, _HID), f32)
    ones = jnp.ones((_LANES, _HID), f32)
    wp = W_perc.reshape(_HID, _HID)
    bp = b_perc.reshape(1, _HID)
    hw = hop_weights.reshape(1, 2)
    gv = gate_vector.reshape(1, 8)

    t0 = pl.pallas_call(
        _tc_prep,
        out_shape=jax.ShapeDtypeStruct((_N_PAD, _HID), f32),
    )(x, W_hop, b_hop)

    part1 = _build_sc_hop1()(t0, srcr2, dstr2, zrows, ones)

    h1, t1 = pl.pallas_call(
        _tc_mid,
        out_shape=[
            jax.ShapeDtypeStruct((_N_PAD, _HID), f32),
            jax.ShapeDtypeStruct((_N_PAD, _HID), f32),
        ],
    )(part1, W_hop, b_hop)

    part2 = _build_sc_pass()(t1, srcr, dstr, zrows)

    out = pl.pallas_call(
        _tc_final,
        out_shape=jax.ShapeDtypeStruct((_N_PAD, _HID), f32),
    )(part2, part1, h1, hw, wp, bp, gv)

    return out[:_N]


# speedup vs baseline: 1.2817x; 1.2817x over previous
import functools

import jax
import jax.numpy as jnp
from jax import lax
from jax.experimental import pallas as pl
from jax.experimental.pallas import tpu as pltpu
from jax.experimental.pallas import tpu_sc as plsc

_N = 10000
_HID = 128
_E = 320000
_NC = 2
_NS = 16
_NW = _NC * _NS
_LANES = 128
_GI = 16
_KG = 5
_K = 80
_E_PAD = _NW * _K * _LANES
_FAST_CID = 1
_KF = 144
_KS = 16
_KGF = _KF // _GI
_KGS = _KS // _GI
_SLOW_CID = 1 - _FAST_CID
_K2 = 2 * _K
_KG2 = _K2 // _GI
_N_PAD = 10240
_RPT = _N_PAD // _NS


def _tc_prep(x_ref, wh_ref, bh_ref, t0_ref):
    wbar = jnp.mean(wh_ref[...], axis=0)
    bbar = jnp.mean(bh_ref[...], axis=0, keepdims=True)
    t0_ref[...] = lax.dot_general(
        x_ref[...], wbar, (((1,), (1,)), ((), ())),
        preferred_element_type=jnp.float32,
        precision=lax.Precision.HIGHEST) + bbar


def _tc_mid(part_ref, degp_ref, wh_ref, bh_ref, h1_ref, t1_ref):
    deg = degp_ref[0, :, 0:1] + degp_ref[1, :, 0:1]
    den = jnp.maximum(deg, 1.0)
    h1 = (part_ref[0] + part_ref[1]) / den
    wbar = jnp.mean(wh_ref[...], axis=0)
    bbar = jnp.mean(bh_ref[...], axis=0, keepdims=True)
    h1_ref[...] = h1
    t1_ref[...] = lax.dot_general(
        h1, wbar, (((1,), (1,)), ((), ())),
        preferred_element_type=jnp.float32,
        precision=lax.Precision.HIGHEST) + bbar


def _tc_final(part2_ref, degp_ref, h1_ref, hw_ref, wp_ref, bp_ref, gv_ref,
              out_ref):
    deg = degp_ref[0, :, 0:1] + degp_ref[1, :, 0:1]
    den = jnp.maximum(deg, 1.0)
    h2 = (part2_ref[0] + part2_ref[1]) / den
    ehw = jnp.exp(hw_ref[...])
    hws = jnp.sum(ehw)
    hf = h1_ref[...] * (ehw[:, 0:1] / hws) + h2 * (ehw[:, 1:2] / hws)
    egv = jnp.exp(gv_ref[...])
    gw = egv / jnp.sum(egv)
    col = lax.broadcasted_iota(jnp.int32, (8, _HID), 1) // 16
    row = lax.broadcasted_iota(jnp.int32, (8, _HID), 0)
    sel = (col == row).astype(jnp.float32)
    gwv = jnp.dot(gw, sel, preferred_element_type=jnp.float32)
    z = lax.dot_general(
        hf, wp_ref[...], (((1,), (1,)), ((), ())),
        preferred_element_type=jnp.float32,
        precision=lax.Precision.HIGHEST) + bp_ref[...]
    out_ref[...] = z * gwv


def _sc_edge_body(table_hbm, srcr_hbm, dstr_hbm, zrows_hbm, part_hbm,
                  acc, src_v, dst_v, rows_a, rows_b, sem_a, sem_b):
    cid = lax.axis_index("c")
    sid = lax.axis_index("s")
    row0 = sid * _RPT

    pltpu.sync_copy(zrows_hbm, rows_a)
    for j in range(_RPT // _LANES):
        pltpu.sync_copy(rows_a, acc.at[pl.ds(row0 + j * _LANES, _LANES)])
    plsc.subcore_barrier()

    bufs = (rows_a, rows_b)
    sems = (sem_a, sem_b)

    def group(g, carry):
        pltpu.sync_copy(srcr_hbm.at[cid, sid, pl.ds(g * _GI, _GI)], src_v)
        pltpu.sync_copy(dstr_hbm.at[cid, sid, pl.ds(g * _GI, _GI)], dst_v)
        cps = {0: pltpu.async_copy(table_hbm.at[src_v.at[0]], bufs[0],
                                   sems[0])}
        for j in range(_GI):
            cps[j].wait()
            if j + 1 < _GI:
                cps[j + 1] = pltpu.async_copy(
                    table_hbm.at[src_v.at[j + 1]], bufs[(j + 1) % 2],
                    sems[(j + 1) % 2])
            pltpu.sync_copy(bufs[j % 2], acc.at[dst_v.at[j]], add=True)
        return carry

    kg = jnp.where(cid == _FAST_CID, _KGF, _KGS)
    lax.fori_loop(0, kg, group, 0)

    plsc.subcore_barrier()
    for j in range(_RPT // _LANES):
        r = row0 + j * _LANES
        pltpu.sync_copy(acc.at[pl.ds(r, _LANES)], rows_a)
        pltpu.sync_copy(rows_a, part_hbm.at[cid, pl.ds(r, _LANES)])


@functools.lru_cache(maxsize=None)
def _build_sc_pass():
    return pl.kernel(
        _sc_edge_body,
        mesh=plsc.VectorSubcoreMesh(core_axis_name="c", subcore_axis_name="s"),
        out_type=jax.ShapeDtypeStruct((_NC, _N_PAD, _HID), jnp.float32),
        scratch_types=[
            pltpu.VMEM_SHARED((_N_PAD, _HID), jnp.float32),
            pltpu.VMEM((_GI, _LANES), jnp.int32),
            pltpu.VMEM((_GI, _LANES), jnp.int32),
            pltpu.VMEM((_LANES, _HID), jnp.float32),
            pltpu.VMEM((_LANES, _HID), jnp.float32),
            pltpu.SemaphoreType.DMA,
            pltpu.SemaphoreType.DMA,
        ],
    )


def _sc_deg_body(dstr_hbm, zrows_hbm, ones_hbm, degp_hbm,
                 dacc, dst_v, buf):
    cid = lax.axis_index("c")
    sid = lax.axis_index("s")
    wid = sid * _NC + cid
    row0 = sid * _RPT

    pltpu.sync_copy(zrows_hbm, buf)
    for j in range(_RPT // _LANES):
        pltpu.sync_copy(buf, dacc.at[pl.ds(row0 + j * _LANES, _LANES)])
    pltpu.sync_copy(ones_hbm, buf)
    plsc.subcore_barrier()

    def group(g, carry):
        pltpu.sync_copy(dstr_hbm.at[wid, pl.ds(g * _GI, _GI)], dst_v)
        for j in range(_GI):
            pltpu.sync_copy(buf, dacc.at[dst_v.at[j]], add=True)
        return carry

    lax.fori_loop(0, _KG, group, 0)

    plsc.subcore_barrier()
    for j in range(_RPT // _LANES):
        r = row0 + j * _LANES
        pltpu.sync_copy(dacc.at[pl.ds(r, _LANES)], buf)
        pltpu.sync_copy(buf, degp_hbm.at[cid, pl.ds(r, _LANES)])


@functools.lru_cache(maxsize=None)
def _build_sc_deg():
    return pl.kernel(
        _sc_deg_body,
        mesh=plsc.VectorSubcoreMesh(core_axis_name="c", subcore_axis_name="s"),
        out_type=jax.ShapeDtypeStruct((_NC, _N_PAD, _HID), jnp.float32),
        scratch_types=[
            pltpu.VMEM_SHARED((_N_PAD, _HID), jnp.float32),
            pltpu.VMEM((_GI, _LANES), jnp.int32),
            pltpu.VMEM((_LANES, _HID), jnp.float32),
        ],
    )


def kernel(node_features, edge_index, W_hop, b_hop, hop_weights, W_perc,
           b_perc, gate_vector):
    f32 = jnp.float32
    x = jnp.pad(node_features, ((0, _N_PAD - _N), (0, 0)))
    src = edge_index[0]
    dst = edge_index[1]
    pad_e = _E_PAD - _E
    src_p = jnp.concatenate([src, jnp.zeros((pad_e,), jnp.int32)])
    dst_p = jnp.concatenate([dst, jnp.full((pad_e,), _N, jnp.int32)])
    dstr_deg = dst_p.reshape(_NW, _K, _LANES)

    def _split(e_p):
        ef = _NS * _KF * _LANES
        fast = e_p[:ef].reshape(_NS, _KF, _LANES)
        slow = jnp.pad(
            e_p[ef:].reshape(_NS, _KS, _LANES),
            ((0, 0), (0, _KF - _KS), (0, 0)))
        cores = [None, None]
        cores[_FAST_CID] = fast
        cores[1 - _FAST_CID] = slow
        return jnp.stack(cores)
    srcr = _split(src_p)
    dstr = _split(dst_p)
    zrows = jnp.zeros((_LANES, _HID), f32)
    ones = jnp.ones((_LANES, _HID), f32)
    wp = W_perc.reshape(_HID, _HID)
    bp = b_perc.reshape(1, _HID)
    hw = hop_weights.reshape(1, 2)
    gv = gate_vector.reshape(1, 8)

    t0 = pl.pallas_call(
        _tc_prep,
        out_shape=jax.ShapeDtypeStruct((_N_PAD, _HID), f32),
    )(x, W_hop, b_hop)

    degp = _build_sc_deg()(dstr_deg, zrows, ones)
    part1 = _build_sc_pass()(t0, srcr, dstr, zrows)

    h1, t1 = pl.pallas_call(
        _tc_mid,
        out_shape=[
            jax.ShapeDtypeStruct((_N_PAD, _HID), f32),
            jax.ShapeDtypeStruct((_N_PAD, _HID), f32),
        ],
    )(part1, degp, W_hop, b_hop)

    part2 = _build_sc_pass()(t1, srcr, dstr, zrows)

    out = pl.pallas_call(
        _tc_final,
        out_shape=jax.ShapeDtypeStruct((_N_PAD, _HID), f32),
    )(part2, degp, h1, hw, wp, bp, gv)

    return out[:_N]


# speedup vs baseline: 1.2894x; 1.0060x over previous
import functools

import jax
import jax.numpy as jnp
from jax import lax
from jax.experimental import pallas as pl
from jax.experimental.pallas import tpu as pltpu
from jax.experimental.pallas import tpu_sc as plsc

_N = 10000
_HID = 128
_E = 320000
_NC = 2
_NS = 16
_NW = _NC * _NS
_LANES = 128
_GI = 16
_KG = 5
_K = 80
_E_PAD = _NW * _K * _LANES
_FAST_CID = 1
_KF = 144
_KS = 16
_KGF = _KF // _GI
_KGS = _KS // _GI
_SLOW_CID = 1 - _FAST_CID
_K2 = 2 * _K
_KG2 = _K2 // _GI
_N_PAD = 10240
_RPT = _N_PAD // _NS
_DEGW = _HID


def _tc_prep(x_ref, wh_ref, bh_ref, t0_ref):
    wbar = jnp.mean(wh_ref[...], axis=0)
    bbar = jnp.mean(bh_ref[...], axis=0, keepdims=True)
    t0_ref[...] = lax.dot_general(
        x_ref[...], wbar, (((1,), (1,)), ((), ())),
        preferred_element_type=jnp.float32,
        precision=lax.Precision.HIGHEST) + bbar


def _tc_mid(part_ref, degp_ref, wh_ref, bh_ref, h1_ref, t1_ref):
    deg = degp_ref[0, :, 0:1] + degp_ref[1, :, 0:1]
    den = jnp.maximum(deg, 1.0)
    h1 = (part_ref[0] + part_ref[1]) / den
    wbar = jnp.mean(wh_ref[...], axis=0)
    bbar = jnp.mean(bh_ref[...], axis=0, keepdims=True)
    h1_ref[...] = h1
    t1_ref[...] = lax.dot_general(
        h1, wbar, (((1,), (1,)), ((), ())),
        preferred_element_type=jnp.float32,
        precision=lax.Precision.HIGHEST) + bbar


def _tc_final(part2_ref, degp_ref, h1_ref, hw_ref, wp_ref, bp_ref, gv_ref,
              out_ref):
    deg = degp_ref[0, :, 0:1] + degp_ref[1, :, 0:1]
    den = jnp.maximum(deg, 1.0)
    h2 = (part2_ref[0] + part2_ref[1]) / den
    ehw = jnp.exp(hw_ref[...])
    hws = jnp.sum(ehw)
    hf = h1_ref[...] * (ehw[:, 0:1] / hws) + h2 * (ehw[:, 1:2] / hws)
    egv = jnp.exp(gv_ref[...])
    gw = egv / jnp.sum(egv)
    col = lax.broadcasted_iota(jnp.int32, (8, _HID), 1) // 16
    row = lax.broadcasted_iota(jnp.int32, (8, _HID), 0)
    sel = (col == row).astype(jnp.float32)
    gwv = jnp.dot(gw, sel, preferred_element_type=jnp.float32)
    z = lax.dot_general(
        hf[:_N], wp_ref[...], (((1,), (1,)), ((), ())),
        preferred_element_type=jnp.float32,
        precision=lax.Precision.HIGHEST) + bp_ref[...]
    out_ref[...] = z * gwv


def _sc_edge_body(table_hbm, srcr_hbm, dstr_hbm, zrows_hbm, part_hbm,
                  acc, src_v, dst_v, rows_a, rows_b, sem_a, sem_b):
    cid = lax.axis_index("c")
    sid = lax.axis_index("s")
    row0 = sid * _RPT

    pltpu.sync_copy(zrows_hbm, rows_a)
    for j in range(_RPT // _LANES):
        pltpu.sync_copy(rows_a, acc.at[pl.ds(row0 + j * _LANES, _LANES)])
    plsc.subcore_barrier()

    bufs = (rows_a, rows_b)
    sems = (sem_a, sem_b)

    def group(g, carry):
        pltpu.sync_copy(srcr_hbm.at[cid, sid, pl.ds(g * _GI, _GI)], src_v)
        pltpu.sync_copy(dstr_hbm.at[cid, sid, pl.ds(g * _GI, _GI)], dst_v)
        cps = {0: pltpu.async_copy(table_hbm.at[src_v.at[0]], bufs[0],
                                   sems[0])}
        for j in range(_GI):
            cps[j].wait()
            if j + 1 < _GI:
                cps[j + 1] = pltpu.async_copy(
                    table_hbm.at[src_v.at[j + 1]], bufs[(j + 1) % 2],
                    sems[(j + 1) % 2])
            pltpu.sync_copy(bufs[j % 2], acc.at[dst_v.at[j]], add=True)
        return carry

    kg = jnp.where(cid == _FAST_CID, _KGF, _KGS)
    lax.fori_loop(0, kg, group, 0)

    plsc.subcore_barrier()
    for j in range(_RPT // _LANES):
        r = row0 + j * _LANES
        pltpu.sync_copy(acc.at[pl.ds(r, _LANES)], rows_a)
        pltpu.sync_copy(rows_a, part_hbm.at[cid, pl.ds(r, _LANES)])


@functools.lru_cache(maxsize=None)
def _build_sc_pass():
    return pl.kernel(
        _sc_edge_body,
        mesh=plsc.VectorSubcoreMesh(core_axis_name="c", subcore_axis_name="s"),
        out_type=jax.ShapeDtypeStruct((_NC, _N_PAD, _HID), jnp.float32),
        scratch_types=[
            pltpu.VMEM_SHARED((_N_PAD, _HID), jnp.float32),
            pltpu.VMEM((_GI, _LANES), jnp.int32),
            pltpu.VMEM((_GI, _LANES), jnp.int32),
            pltpu.VMEM((_LANES, _HID), jnp.float32),
            pltpu.VMEM((_LANES, _HID), jnp.float32),
            pltpu.SemaphoreType.DMA,
            pltpu.SemaphoreType.DMA,
        ],
    )


def _sc_deg_body(dstr_hbm, zrows_hbm, ones_hbm, degp_hbm,
                 dacc, dst_v, buf):
    cid = lax.axis_index("c")
    sid = lax.axis_index("s")
    wid = sid * _NC + cid
    row0 = sid * _RPT

    pltpu.sync_copy(zrows_hbm, buf)
    for j in range(_RPT // _LANES):
        pltpu.sync_copy(buf, dacc.at[pl.ds(row0 + j * _LANES, _LANES)])
    pltpu.sync_copy(ones_hbm, buf)
    plsc.subcore_barrier()

    def group(g, carry):
        pltpu.sync_copy(dstr_hbm.at[wid, pl.ds(g * _GI, _GI)], dst_v)
        for j in range(_GI):
            pltpu.sync_copy(buf, dacc.at[dst_v.at[j]], add=True)
        return carry

    lax.fori_loop(0, _KG, group, 0)

    plsc.subcore_barrier()
    for j in range(_RPT // _LANES):
        r = row0 + j * _LANES
        pltpu.sync_copy(dacc.at[pl.ds(r, _LANES)], buf)
        pltpu.sync_copy(buf, degp_hbm.at[cid, pl.ds(r, _LANES)])


@functools.lru_cache(maxsize=None)
def _build_sc_deg():
    return pl.kernel(
        _sc_deg_body,
        mesh=plsc.VectorSubcoreMesh(core_axis_name="c", subcore_axis_name="s"),
        out_type=jax.ShapeDtypeStruct((_NC, _N_PAD, _DEGW), jnp.float32),
        scratch_types=[
            pltpu.VMEM_SHARED((_N_PAD, _DEGW), jnp.float32),
            pltpu.VMEM((_GI, _LANES), jnp.int32),
            pltpu.VMEM((_LANES, _DEGW), jnp.float32),
        ],
    )


def kernel(node_features, edge_index, W_hop, b_hop, hop_weights, W_perc,
           b_perc, gate_vector):
    f32 = jnp.float32
    x = jnp.pad(node_features, ((0, _N_PAD - _N), (0, 0)))
    src = edge_index[0]
    dst = edge_index[1]
    pad_e = _E_PAD - _E
    src_p = jnp.concatenate([src, jnp.zeros((pad_e,), jnp.int32)])
    dst_p = jnp.concatenate([dst, jnp.full((pad_e,), _N, jnp.int32)])
    dstr_deg = dst_p.reshape(_NW, _K, _LANES)

    def _split(e_p):
        ef = _NS * _KF * _LANES
        fast = e_p[:ef].reshape(_NS, _KF, _LANES)
        slow = jnp.pad(
            e_p[ef:].reshape(_NS, _KS, _LANES),
            ((0, 0), (0, _KF - _KS), (0, 0)))
        cores = [None, None]
        cores[_FAST_CID] = fast
        cores[1 - _FAST_CID] = slow
        return jnp.stack(cores)
    srcr = _split(src_p)
    dstr = _split(dst_p)
    zrows = jnp.zeros((_LANES, _HID), f32)
    zdeg = jnp.zeros((_LANES, _DEGW), f32)
    ones = jnp.ones((_LANES, _DEGW), f32)
    wp = W_perc.reshape(_HID, _HID)
    bp = b_perc.reshape(1, _HID)
    hw = hop_weights.reshape(1, 2)
    gv = gate_vector.reshape(1, 8)

    t0 = pl.pallas_call(
        _tc_prep,
        out_shape=jax.ShapeDtypeStruct((_N_PAD, _HID), f32),
    )(x, W_hop, b_hop)

    degp = _build_sc_deg()(dstr_deg, zdeg, ones)
    part1 = _build_sc_pass()(t0, srcr, dstr, zrows)

    h1, t1 = pl.pallas_call(
        _tc_mid,
        out_shape=[
            jax.ShapeDtypeStruct((_N_PAD, _HID), f32),
            jax.ShapeDtypeStruct((_N_PAD, _HID), f32),
        ],
    )(part1, degp, W_hop, b_hop)

    part2 = _build_sc_pass()(t1, srcr, dstr, zrows)

    out = pl.pallas_call(
        _tc_final,
        out_shape=jax.ShapeDtypeStruct((_N, _HID), f32),
    )(part2, degp, h1, hw, wp, bp, gv)

    return out
